# baseline (speedup 1.0000x reference)
import functools

import jax
import jax.numpy as jnp
from jax import lax
from jax.experimental import pallas as pl
from jax.experimental.pallas import tpu as pltpu
from jax.experimental.pallas import tpu_sc as plsc

_N = 10000
_E = 320000
_D = 128
_L = 3
_G = 64

_R = 1000
_GRID = _N // _R

_NC = 2
_NS = 16
_NW = _NC * _NS
_C = 40
_EPW = _E // _NW
_CHUNKS = _EPW // _C
_CB = 80
_NB = _N // _CB


def _enc_body(x_ref, w_ref, b_ref, o_ref):
    o_ref[...] = (
        jnp.dot(x_ref[...], w_ref[...], preferred_element_type=jnp.float32)
        + b_ref[...]
    )


def _linear(x, wt, b):
    return pl.pallas_call(
        _enc_body,
        grid=(_GRID,),
        in_specs=[
            pl.BlockSpec((_R, _D), lambda i: (i, 0)),
            pl.BlockSpec((_D, _D), lambda i: (0, 0)),
            pl.BlockSpec((1, _D), lambda i: (0, 0)),
        ],
        out_specs=pl.BlockSpec((_R, _D), lambda i: (i, 0)),
        out_shape=jax.ShapeDtypeStruct((_N, _D), jnp.float32),
    )(x, wt, b.reshape(1, _D))


def _kqv_body(h_ref, wk_ref, bk_ref, wqv_ref, bqv_ref, k_ref, qv_ref):
    hb = h_ref[...]
    k_ref[...] = (
        jnp.dot(hb, wk_ref[...], preferred_element_type=jnp.float32)
        + bk_ref[...]
    )
    qv_ref[...] = (
        jnp.dot(hb, wqv_ref[...], preferred_element_type=jnp.float32)
        + bqv_ref[...]
    )


def _kqv(h, wkt, bk, wqvt, bqv):
    return pl.pallas_call(
        _kqv_body,
        grid=(_GRID,),
        in_specs=[
            pl.BlockSpec((_R, _D), lambda i: (i, 0)),
            pl.BlockSpec((_D, _D), lambda i: (0, 0)),
            pl.BlockSpec((1, _D), lambda i: (0, 0)),
            pl.BlockSpec((_D, 2 * _D), lambda i: (0, 0)),
            pl.BlockSpec((1, 2 * _D), lambda i: (0, 0)),
        ],
        out_specs=[
            pl.BlockSpec((_R, _D), lambda i: (i, 0)),
            pl.BlockSpec((_R, 2 * _D), lambda i: (i, 0)),
        ],
        out_shape=[
            jax.ShapeDtypeStruct((_N, _D), jnp.float32),
            jax.ShapeDtypeStruct((_N, 2 * _D), jnp.float32),
        ],
    )(h, wkt, bk.reshape(1, _D), wqvt, bqv.reshape(1, 2 * _D))


def _out_body(h_ref, ws_ref, b_ref, a_ref, p_ref, o_ref):
    hb = h_ref[...]
    aggr = jnp.dot(
        a_ref[0] + a_ref[1], p_ref[...], preferred_element_type=jnp.float32
    )
    t = (
        jnp.dot(hb, ws_ref[...], preferred_element_type=jnp.float32)
        + b_ref[...]
        + aggr
    )
    o_ref[...] = jnp.where(t >= 0.0, t, 0.01 * t) + hb


def _layer_out(h, wst, b, parts, perm):
    return pl.pallas_call(
        _out_body,
        grid=(_GRID,),
        in_specs=[
            pl.BlockSpec((_R, _D), lambda i: (i, 0)),
            pl.BlockSpec((_D, _D), lambda i: (0, 0)),
            pl.BlockSpec((1, _D), lambda i: (0, 0)),
            pl.BlockSpec((2, _R, _D), lambda i: (0, i, 0)),
            pl.BlockSpec((_D, _D), lambda i: (0, 0)),
        ],
        out_specs=pl.BlockSpec((_R, _D), lambda i: (i, 0)),
        out_shape=jax.ShapeDtypeStruct((_N, _D), jnp.float32),
    )(h, wst, b.reshape(1, _D), parts, perm)


def _pool_body(b_ref, h_ref, o_ref, sums, counts):
    i = pl.program_id(0)

    @pl.when(i == 0)
    def _():
        sums[...] = jnp.zeros_like(sums)
        counts[...] = jnp.zeros_like(counts)

    seg = b_ref[0]
    onehot = (
        seg[None, :] == lax.broadcasted_iota(jnp.int32, (_G, _R), 0)
    ).astype(jnp.float32)
    sums[...] += jnp.dot(onehot, h_ref[...], preferred_element_type=jnp.float32)
    counts[...] += jnp.sum(onehot, axis=1, keepdims=True)

    @pl.when(i == pl.num_programs(0) - 1)
    def _():
        o_ref[...] = sums[...] / jnp.maximum(counts[...], 1.0)


def _pool(h, batch32):
    return pl.pallas_call(
        _pool_body,
        grid=(_GRID,),
        in_specs=[
            pl.BlockSpec((None, 1, _R), lambda i: (i, 0, 0)),
            pl.BlockSpec((_R, _D), lambda i: (i, 0)),
        ],
        out_specs=pl.BlockSpec((_G, _D), lambda i: (0, 0)),
        out_shape=jax.ShapeDtypeStruct((_G, _D), jnp.float32),
        scratch_shapes=[
            pltpu.VMEM((_G, _D), jnp.float32),
            pltpu.VMEM((_G, _D), jnp.float32),
        ],
    )(batch32.reshape(_GRID, 1, _R), h)


def _sc_edge_body(k_hbm, qv_hbm, src_hbm, dst_hbm, out_hbm,
                  isl0, dsa0, dsb0, kb0, qvb0, mb0,
                  isl1, dsa1, dsb1, kb1, qvb1, mb1,
                  isl2, dsa2, dsb2, kb2, qvb2, mb2, acc,
                  gk0, gq0, is0, ss0, gk1, gq1, is1, ss1,
                  gk2, gq2, is2, ss2):
    c = lax.axis_index("c")
    s = lax.axis_index("s")
    wid = s * _NC + c
    base0 = wid * _EPW

    def fetch_idx(t, isl, dsc, isem):
        pltpu.async_copy(src_hbm.at[pl.ds(base0 + t * _C, _C)], isl, isem)
        pltpu.async_copy(dst_hbm.at[pl.ds(base0 + t * _C, _C)], dsc, isem)

    def wait_idx(t, isl, dsc, isem):
        pltpu.make_async_copy(
            src_hbm.at[pl.ds(base0 + t * _C, _C)], isl, isem).wait()
        pltpu.make_async_copy(
            dst_hbm.at[pl.ds(base0 + t * _C, _C)], dsc, isem).wait()

    def gathers(isl, dsc, kb, qvb, gks, gqs):
        pltpu.async_copy(k_hbm.at[dsc], kb, gks)
        pltpu.async_copy(qv_hbm.at[isl], qvb, gqs)

    fetch_idx(0, isl0, dsa0, is0)
    fetch_idx(1, isl1, dsa1, is1)
    fetch_idx(2, isl2, dsa2, is2)
    wait_idx(0, isl0, dsa0, is0)
    gathers(isl0, dsa0, kb0, qvb0, gk0, gq0)
    wait_idx(1, isl1, dsa1, is1)
    gathers(isl1, dsa1, kb1, qvb1, gk1, gq1)
    wait_idx(2, isl2, dsa2, is2)
    gathers(isl2, dsa2, kb2, qvb2, gk2, gq2)

    def zrow(i, carry):
        for j in range(_D // 16):
            mb0[i, pl.ds(j * 16, 16)] = jnp.zeros((16,), jnp.float32)
        return carry

    lax.fori_loop(0, _C, zrow, 0)

    for j in range(_NB // _NS + 1):
        bi = j * _NS + s

        @pl.when(bi < _NB)
        def _():
            for r in range(_CB // _C):
                pltpu.sync_copy(mb0, acc.at[pl.ds(bi * _CB + r * _C, _C)])

    plsc.subcore_barrier()

    def step(t, isl, dsc, dsp, kb, qvb, mb, gks, gqs, sss, isem):
        pltpu.make_async_copy(k_hbm.at[dsc], kb, gks).wait()
        pltpu.make_async_copy(qv_hbm.at[isl], qvb, gqs).wait()

        @pl.when(t >= 3)
        def _():
            pltpu.make_async_copy(mb, acc.at[dsp], sss).wait()

        @pl.when(t + 3 < _CHUNKS)
        def _():
            fetch_idx(t + 3, isl, dsp, isem)

        @plsc.parallel_loop(0, _C, unroll=4)
        def _(i):
            for m in range(_D // 32):
                wq = qvb[i, pl.ds(16 * m, 16)]
                wv = qvb[i, pl.ds(_D // 2 + 16 * m, 16)]
                q0 = lax.bitcast_convert_type(wq << 16, jnp.float32)
                q1 = lax.bitcast_convert_type(
                    wq & jnp.int32(-65536), jnp.float32)
                v0 = lax.bitcast_convert_type(wv << 16, jnp.float32)
                v1 = lax.bitcast_convert_type(
                    wv & jnp.int32(-65536), jnp.float32)
                sl0 = pl.ds(32 * m, 16)
                sl1 = pl.ds(32 * m + 16, 16)
                x0 = kb[i, sl0] + q0
                x1 = kb[i, sl1] + q1
                mb[i, sl0] = v0 / (1.0 + jnp.exp(-x0))
                mb[i, sl1] = v1 / (1.0 + jnp.exp(-x1))

        pltpu.async_copy(mb, acc.at[dsc], sss, add=True)

        @pl.when(t + 3 < _CHUNKS)
        def _():
            wait_idx(t + 3, isl, dsp, isem)
            gathers(isl, dsp, kb, qvb, gks, gqs)

    def hexa(i, carry):
        t0 = 6 * i
        step(t0, isl0, dsa0, dsb0, kb0, qvb0, mb0, gk0, gq0, ss0, is0)
        step(t0 + 1, isl1, dsa1, dsb1, kb1, qvb1, mb1, gk1, gq1, ss1, is1)
        step(t0 + 2, isl2, dsa2, dsb2, kb2, qvb2, mb2, gk2, gq2, ss2, is2)
        step(t0 + 3, isl0, dsb0, dsa0, kb0, qvb0, mb0, gk0, gq0, ss0, is0)
        step(t0 + 4, isl1, dsb1, dsa1, kb1, qvb1, mb1, gk1, gq1, ss1, is1)
        step(t0 + 5, isl2, dsb2, dsa2, kb2, qvb2, mb2, gk2, gq2, ss2, is2)
        return carry

    lax.fori_loop(0, (_CHUNKS - 4) // 6, hexa, 0)
    step(_CHUNKS - 4, isl0, dsa0, dsb0, kb0, qvb0, mb0, gk0, gq0, ss0, is0)
    step(_CHUNKS - 3, isl1, dsa1, dsb1, kb1, qvb1, mb1, gk1, gq1, ss1, is1)
    step(_CHUNKS - 2, isl2, dsa2, dsb2, kb2, qvb2, mb2, gk2, gq2, ss2, is2)
    step(_CHUNKS - 1, isl0, dsb0, dsa0, kb0, qvb0, mb0, gk0, gq0, ss0, is0)
    pltpu.make_async_copy(mb1, acc.at[dsa1], ss1).wait()
    pltpu.make_async_copy(mb2, acc.at[dsa2], ss2).wait()
    pltpu.make_async_copy(mb0, acc.at[dsb0], ss0).wait()
    plsc.subcore_barrier()

    for j in range(_NB // _NS + 1):
        bi = j * _NS + s

        @pl.when(bi < _NB)
        def _():
            pltpu.sync_copy(
                acc.at[pl.ds(bi * _CB, _CB)],
                out_hbm.at[pl.ds(c * _N + bi * _CB, _CB)],
            )


_sc_edge = functools.partial(
    pl.kernel,
    out_type=jax.ShapeDtypeStruct((_NC * _N, _D), jnp.float32),
    mesh=plsc.VectorSubcoreMesh(core_axis_name="c", subcore_axis_name="s"),
    scratch_types=(
        [
            pltpu.VMEM((_C,), jnp.int32),
            pltpu.VMEM((_C,), jnp.int32),
            pltpu.VMEM((_C,), jnp.int32),
            pltpu.VMEM((_C, _D), jnp.float32),
            pltpu.VMEM((_C, _D), jnp.int32),
            pltpu.VMEM((_C, _D), jnp.float32),
        ] * 3
        + [pltpu.VMEM_SHARED((_N, _D), jnp.float32)]
        + [pltpu.SemaphoreType.DMA] * 12
    ),
)(_sc_edge_body)


def kernel(x, edge_index, batch, W_enc, b_enc, Wk, bk, Wq, bq, Wv, bv,
           Wskip, bias):
    src = edge_index[0].astype(jnp.int32)
    dst = edge_index[1].astype(jnp.int32)
    batch32 = batch.astype(jnp.int32)

    tau = (
        jnp.arange(_D, dtype=jnp.int32)
        .reshape(_D // 32, 16, 2)
        .swapaxes(1, 2)
        .reshape(_D)
    )
    perm = jnp.eye(_D, dtype=jnp.float32)[tau]

    h = _linear(x, W_enc.T, b_enc)
    for l in range(_L):
        wqvt = jnp.concatenate([Wq[l].T, Wv[l].T], axis=1)
        bqv = jnp.concatenate([bq[l], bv[l]])
        k, qv = _kqv(h, Wk[l].T[:, tau], bk[l][tau], wqvt, bqv)
        qvp = lax.bitcast_convert_type(
            qv.astype(jnp.bfloat16).reshape(_N, _D, 2), jnp.int32
        )
        parts = _sc_edge(k, qvp, src, dst)
        h = _layer_out(
            h, Wskip[l].T, bias[l], parts.reshape(_NC, _N, _D), perm
        )
    return _pool(h, batch32)


# speedup vs baseline: 1.0950x; 1.0950x over previous
import functools

import jax
import jax.numpy as jnp
from jax import lax
from jax.experimental import pallas as pl
from jax.experimental.pallas import tpu as pltpu
from jax.experimental.pallas import tpu_sc as plsc

_N = 10000
_E = 320000
_D = 128
_L = 3
_G = 64

_R = 1000
_GRID = _N // _R

_NC = 2
_NS = 16
_NW = _NC * _NS
_C = 40
_EPW = _E // _NW
_CHUNKS = _EPW // _C
_CB = 80
_NB = _N // _CB


def _enc_body(x_ref, w_ref, b_ref, o_ref):
    o_ref[...] = (
        jnp.dot(x_ref[...], w_ref[...], preferred_element_type=jnp.float32)
        + b_ref[...]
    )


def _linear(x, wt, b):
    return pl.pallas_call(
        _enc_body,
        grid=(_GRID,),
        in_specs=[
            pl.BlockSpec((_R, _D), lambda i: (i, 0)),
            pl.BlockSpec((_D, _D), lambda i: (0, 0)),
            pl.BlockSpec((1, _D), lambda i: (0, 0)),
        ],
        out_specs=pl.BlockSpec((_R, _D), lambda i: (i, 0)),
        out_shape=jax.ShapeDtypeStruct((_N, _D), jnp.float32),
    )(x, wt, b.reshape(1, _D))


def _pack_bf16_pairs(x):
    xb = x.astype(jnp.bfloat16)
    xu = lax.bitcast_convert_type(xb, jnp.uint16).astype(jnp.uint32)
    xr = xu.reshape(x.shape[0], _D // 32, 2, 16)
    w = xr[:, :, 0, :] | (xr[:, :, 1, :] << 16)
    return lax.bitcast_convert_type(w.reshape(x.shape[0], _D // 2), jnp.int32)


def _kqv_body(h_ref, wk_ref, bk_ref, wqv_ref, bqv_ref, k_ref, qv_ref):
    hb = h_ref[...]
    k_ref[...] = (
        jnp.dot(hb, wk_ref[...], preferred_element_type=jnp.float32)
        + bk_ref[...]
    )
    qv = (
        jnp.dot(hb, wqv_ref[...], preferred_element_type=jnp.float32)
        + bqv_ref[...]
    )
    qv_ref[...] = jnp.concatenate(
        [_pack_bf16_pairs(qv[:, :_D]), _pack_bf16_pairs(qv[:, _D:])], axis=1
    )


def _kqv(h, wkt, bk, wqvt, bqv):
    return pl.pallas_call(
        _kqv_body,
        grid=(_GRID,),
        in_specs=[
            pl.BlockSpec((_R, _D), lambda i: (i, 0)),
            pl.BlockSpec((_D, _D), lambda i: (0, 0)),
            pl.BlockSpec((1, _D), lambda i: (0, 0)),
            pl.BlockSpec((_D, 2 * _D), lambda i: (0, 0)),
            pl.BlockSpec((1, 2 * _D), lambda i: (0, 0)),
        ],
        out_specs=[
            pl.BlockSpec((_R, _D), lambda i: (i, 0)),
            pl.BlockSpec((_R, _D), lambda i: (i, 0)),
        ],
        out_shape=[
            jax.ShapeDtypeStruct((_N, _D), jnp.float32),
            jax.ShapeDtypeStruct((_N, _D), jnp.int32),
        ],
    )(h, wkt, bk.reshape(1, _D), wqvt, bqv.reshape(1, 2 * _D))


def _out_body(h_ref, ws_ref, b_ref, a_ref, o_ref):
    hb = h_ref[...]
    t = (
        jnp.dot(hb, ws_ref[...], preferred_element_type=jnp.float32)
        + b_ref[...]
        + a_ref[0]
        + a_ref[1]
    )
    o_ref[...] = jnp.where(t >= 0.0, t, 0.01 * t) + hb


def _layer_out(h, wst, b, parts):
    return pl.pallas_call(
        _out_body,
        grid=(_GRID,),
        in_specs=[
            pl.BlockSpec((_R, _D), lambda i: (i, 0)),
            pl.BlockSpec((_D, _D), lambda i: (0, 0)),
            pl.BlockSpec((1, _D), lambda i: (0, 0)),
            pl.BlockSpec((2, _R, _D), lambda i: (0, i, 0)),
        ],
        out_specs=pl.BlockSpec((_R, _D), lambda i: (i, 0)),
        out_shape=jax.ShapeDtypeStruct((_N, _D), jnp.float32),
    )(h, wst, b.reshape(1, _D), parts)


def _pool_body(b_ref, h_ref, o_ref, sums, counts):
    i = pl.program_id(0)

    @pl.when(i == 0)
    def _():
        sums[...] = jnp.zeros_like(sums)
        counts[...] = jnp.zeros_like(counts)

    seg = b_ref[0]
    onehot = (
        seg[None, :] == lax.broadcasted_iota(jnp.int32, (_G, _R), 0)
    ).astype(jnp.float32)
    sums[...] += jnp.dot(onehot, h_ref[...], preferred_element_type=jnp.float32)
    counts[...] += jnp.sum(onehot, axis=1, keepdims=True)

    @pl.when(i == pl.num_programs(0) - 1)
    def _():
        o_ref[...] = sums[...] / jnp.maximum(counts[...], 1.0)


def _pool(h, batch32):
    return pl.pallas_call(
        _pool_body,
        grid=(_GRID,),
        in_specs=[
            pl.BlockSpec((None, 1, _R), lambda i: (i, 0, 0)),
            pl.BlockSpec((_R, _D), lambda i: (i, 0)),
        ],
        out_specs=pl.BlockSpec((_G, _D), lambda i: (0, 0)),
        out_shape=jax.ShapeDtypeStruct((_G, _D), jnp.float32),
        scratch_shapes=[
            pltpu.VMEM((_G, _D), jnp.float32),
            pltpu.VMEM((_G, _D), jnp.float32),
        ],
    )(batch32.reshape(_GRID, 1, _R), h)


def _enc_kqv_body(x_ref, we_ref, be_ref, wk_ref, bk_ref, wqv_ref, bqv_ref,
                  k_ref, qv_ref, h_ref):
    hb = (
        jnp.dot(x_ref[...], we_ref[...], preferred_element_type=jnp.float32)
        + be_ref[...]
    )
    h_ref[...] = hb
    k_ref[...] = (
        jnp.dot(hb, wk_ref[...], preferred_element_type=jnp.float32)
        + bk_ref[...]
    )
    qv = (
        jnp.dot(hb, wqv_ref[...], preferred_element_type=jnp.float32)
        + bqv_ref[...]
    )
    qv_ref[...] = jnp.concatenate(
        [_pack_bf16_pairs(qv[:, :_D]), _pack_bf16_pairs(qv[:, _D:])], axis=1
    )


def _enc_kqv(x, wet, be, wkt, bk, wqvt, bqv):
    return pl.pallas_call(
        _enc_kqv_body,
        grid=(_GRID,),
        in_specs=[
            pl.BlockSpec((_R, _D), lambda i: (i, 0)),
            pl.BlockSpec((_D, _D), lambda i: (0, 0)),
            pl.BlockSpec((1, _D), lambda i: (0, 0)),
            pl.BlockSpec((_D, _D), lambda i: (0, 0)),
            pl.BlockSpec((1, _D), lambda i: (0, 0)),
            pl.BlockSpec((_D, 2 * _D), lambda i: (0, 0)),
            pl.BlockSpec((1, 2 * _D), lambda i: (0, 0)),
        ],
        out_specs=[
            pl.BlockSpec((_R, _D), lambda i: (i, 0)),
            pl.BlockSpec((_R, _D), lambda i: (i, 0)),
            pl.BlockSpec((_R, _D), lambda i: (i, 0)),
        ],
        out_shape=[
            jax.ShapeDtypeStruct((_N, _D), jnp.float32),
            jax.ShapeDtypeStruct((_N, _D), jnp.int32),
            jax.ShapeDtypeStruct((_N, _D), jnp.float32),
        ],
    )(x, wet, be.reshape(1, _D), wkt, bk.reshape(1, _D), wqvt,
      bqv.reshape(1, 2 * _D))


def _out_kqv_body(h_ref, ws_ref, b_ref, a_ref, wk_ref, bk_ref, wqv_ref,
                  bqv_ref, k_ref, qv_ref, h2_ref):
    hb = h_ref[...]
    t = (
        jnp.dot(hb, ws_ref[...], preferred_element_type=jnp.float32)
        + b_ref[...]
        + a_ref[0]
        + a_ref[1]
    )
    hn = jnp.where(t >= 0.0, t, 0.01 * t) + hb
    h2_ref[...] = hn
    k_ref[...] = (
        jnp.dot(hn, wk_ref[...], preferred_element_type=jnp.float32)
        + bk_ref[...]
    )
    qv = (
        jnp.dot(hn, wqv_ref[...], preferred_element_type=jnp.float32)
        + bqv_ref[...]
    )
    qv_ref[...] = jnp.concatenate(
        [_pack_bf16_pairs(qv[:, :_D]), _pack_bf16_pairs(qv[:, _D:])], axis=1
    )


def _out_kqv(h, wst, b, parts, wkt, bk, wqvt, bqv):
    return pl.pallas_call(
        _out_kqv_body,
        grid=(_GRID,),
        in_specs=[
            pl.BlockSpec((_R, _D), lambda i: (i, 0)),
            pl.BlockSpec((_D, _D), lambda i: (0, 0)),
            pl.BlockSpec((1, _D), lambda i: (0, 0)),
            pl.BlockSpec((2, _R, _D), lambda i: (0, i, 0)),
            pl.BlockSpec((_D, _D), lambda i: (0, 0)),
            pl.BlockSpec((1, _D), lambda i: (0, 0)),
            pl.BlockSpec((_D, 2 * _D), lambda i: (0, 0)),
            pl.BlockSpec((1, 2 * _D), lambda i: (0, 0)),
        ],
        out_specs=[
            pl.BlockSpec((_R, _D), lambda i: (i, 0)),
            pl.BlockSpec((_R, _D), lambda i: (i, 0)),
            pl.BlockSpec((_R, _D), lambda i: (i, 0)),
        ],
        out_shape=[
            jax.ShapeDtypeStruct((_N, _D), jnp.float32),
            jax.ShapeDtypeStruct((_N, _D), jnp.int32),
            jax.ShapeDtypeStruct((_N, _D), jnp.float32),
        ],
    )(h, wst, b.reshape(1, _D), parts, wkt, bk.reshape(1, _D), wqvt,
      bqv.reshape(1, 2 * _D))


def _out_pool_body(h_ref, ws_ref, b_ref, a_ref, bt_ref, o_ref, sums, counts):
    i = pl.program_id(0)

    @pl.when(i == 0)
    def _():
        sums[...] = jnp.zeros_like(sums)
        counts[...] = jnp.zeros_like(counts)

    hb = h_ref[...]
    t = (
        jnp.dot(hb, ws_ref[...], preferred_element_type=jnp.float32)
        + b_ref[...]
        + a_ref[0]
        + a_ref[1]
    )
    hn = jnp.where(t >= 0.0, t, 0.01 * t) + hb
    seg = bt_ref[0]
    onehot = (
        seg[None, :] == lax.broadcasted_iota(jnp.int32, (_G, _R), 0)
    ).astype(jnp.float32)
    sums[...] += jnp.dot(onehot, hn, preferred_element_type=jnp.float32)
    counts[...] += jnp.sum(onehot, axis=1, keepdims=True)

    @pl.when(i == pl.num_programs(0) - 1)
    def _():
        o_ref[...] = sums[...] / jnp.maximum(counts[...], 1.0)


def _out_pool(h, wst, b, parts, batch32):
    return pl.pallas_call(
        _out_pool_body,
        grid=(_GRID,),
        in_specs=[
            pl.BlockSpec((_R, _D), lambda i: (i, 0)),
            pl.BlockSpec((_D, _D), lambda i: (0, 0)),
            pl.BlockSpec((1, _D), lambda i: (0, 0)),
            pl.BlockSpec((2, _R, _D), lambda i: (0, i, 0)),
            pl.BlockSpec((None, 1, _R), lambda i: (i, 0, 0)),
        ],
        out_specs=pl.BlockSpec((_G, _D), lambda i: (0, 0)),
        out_shape=jax.ShapeDtypeStruct((_G, _D), jnp.float32),
        scratch_shapes=[
            pltpu.VMEM((_G, _D), jnp.float32),
            pltpu.VMEM((_G, _D), jnp.float32),
        ],
    )(h, wst, b.reshape(1, _D), parts, batch32.reshape(_GRID, 1, _R))


def _sc_edge_body(k_hbm, qv_hbm, src_hbm, dst_hbm, out_hbm,
                  isl0, dsa0, dsb0, kb0, qvb0, mb0,
                  isl1, dsa1, dsb1, kb1, qvb1, mb1,
                  isl2, dsa2, dsb2, kb2, qvb2, mb2, acc,
                  gk0, gq0, is0, ss0, gk1, gq1, is1, ss1,
                  gk2, gq2, is2, ss2):
    c = lax.axis_index("c")
    s = lax.axis_index("s")
    wid = s * _NC + c
    base0 = wid * _EPW

    def fetch_idx(t, isl, dsc, isem):
        pltpu.async_copy(src_hbm.at[pl.ds(base0 + t * _C, _C)], isl, isem)
        pltpu.async_copy(dst_hbm.at[pl.ds(base0 + t * _C, _C)], dsc, isem)

    def wait_idx(t, isl, dsc, isem):
        pltpu.make_async_copy(
            src_hbm.at[pl.ds(base0 + t * _C, _C)], isl, isem).wait()
        pltpu.make_async_copy(
            dst_hbm.at[pl.ds(base0 + t * _C, _C)], dsc, isem).wait()

    def gathers(isl, dsc, kb, qvb, gks, gqs):
        pltpu.async_copy(k_hbm.at[dsc], kb, gks)
        pltpu.async_copy(qv_hbm.at[isl], qvb, gqs)

    fetch_idx(0, isl0, dsa0, is0)
    fetch_idx(1, isl1, dsa1, is1)
    fetch_idx(2, isl2, dsa2, is2)
    wait_idx(0, isl0, dsa0, is0)
    gathers(isl0, dsa0, kb0, qvb0, gk0, gq0)
    wait_idx(1, isl1, dsa1, is1)
    gathers(isl1, dsa1, kb1, qvb1, gk1, gq1)
    wait_idx(2, isl2, dsa2, is2)
    gathers(isl2, dsa2, kb2, qvb2, gk2, gq2)

    def zrow(i, carry):
        for j in range(_D // 16):
            mb0[i, pl.ds(j * 16, 16)] = jnp.zeros((16,), jnp.float32)
        return carry

    lax.fori_loop(0, _C, zrow, 0)

    for j in range(_NB // _NS + 1):
        bi = j * _NS + s

        @pl.when(bi < _NB)
        def _():
            for r in range(_CB // _C):
                pltpu.sync_copy(mb0, acc.at[pl.ds(bi * _CB + r * _C, _C)])

    plsc.subcore_barrier()

    def step(t, isl, dsc, dsp, kb, qvb, mb, gks, gqs, sss, isem):
        pltpu.make_async_copy(k_hbm.at[dsc], kb, gks).wait()
        pltpu.make_async_copy(qv_hbm.at[isl], qvb, gqs).wait()

        @pl.when(t >= 3)
        def _():
            pltpu.make_async_copy(mb, acc.at[dsp], sss).wait()

        @pl.when(t + 3 < _CHUNKS)
        def _():
            fetch_idx(t + 3, isl, dsp, isem)

        @plsc.parallel_loop(0, _C, unroll=4)
        def _(i):
            for m in range(_D // 32):
                wq = qvb[i, pl.ds(16 * m, 16)]
                wv = qvb[i, pl.ds(_D // 2 + 16 * m, 16)]
                q0 = lax.bitcast_convert_type(wq << 16, jnp.float32)
                q1 = lax.bitcast_convert_type(
                    wq & jnp.int32(-65536), jnp.float32)
                v0 = lax.bitcast_convert_type(wv << 16, jnp.float32)
                v1 = lax.bitcast_convert_type(
                    wv & jnp.int32(-65536), jnp.float32)
                sl0 = pl.ds(32 * m, 16)
                sl1 = pl.ds(32 * m + 16, 16)
                x0 = kb[i, sl0] + q0
                x1 = kb[i, sl1] + q1
                mb[i, sl0] = v0 / (1.0 + jnp.exp(-x0))
                mb[i, sl1] = v1 / (1.0 + jnp.exp(-x1))

        pltpu.async_copy(mb, acc.at[dsc], sss, add=True)

        @pl.when(t + 3 < _CHUNKS)
        def _():
            wait_idx(t + 3, isl, dsp, isem)
            gathers(isl, dsp, kb, qvb, gks, gqs)

    def hexa(i, carry):
        t0 = 6 * i
        step(t0, isl0, dsa0, dsb0, kb0, qvb0, mb0, gk0, gq0, ss0, is0)
        step(t0 + 1, isl1, dsa1, dsb1, kb1, qvb1, mb1, gk1, gq1, ss1, is1)
        step(t0 + 2, isl2, dsa2, dsb2, kb2, qvb2, mb2, gk2, gq2, ss2, is2)
        step(t0 + 3, isl0, dsb0, dsa0, kb0, qvb0, mb0, gk0, gq0, ss0, is0)
        step(t0 + 4, isl1, dsb1, dsa1, kb1, qvb1, mb1, gk1, gq1, ss1, is1)
        step(t0 + 5, isl2, dsb2, dsa2, kb2, qvb2, mb2, gk2, gq2, ss2, is2)
        return carry

    lax.fori_loop(0, (_CHUNKS - 4) // 6, hexa, 0)
    step(_CHUNKS - 4, isl0, dsa0, dsb0, kb0, qvb0, mb0, gk0, gq0, ss0, is0)
    step(_CHUNKS - 3, isl1, dsa1, dsb1, kb1, qvb1, mb1, gk1, gq1, ss1, is1)
    step(_CHUNKS - 2, isl2, dsa2, dsb2, kb2, qvb2, mb2, gk2, gq2, ss2, is2)
    step(_CHUNKS - 1, isl0, dsb0, dsa0, kb0, qvb0, mb0, gk0, gq0, ss0, is0)
    pltpu.make_async_copy(mb1, acc.at[dsa1], ss1).wait()
    pltpu.make_async_copy(mb2, acc.at[dsa2], ss2).wait()
    pltpu.make_async_copy(mb0, acc.at[dsb0], ss0).wait()
    plsc.subcore_barrier()

    for j in range(_NB // _NS + 1):
        bi = j * _NS + s

        @pl.when(bi < _NB)
        def _():
            pltpu.sync_copy(
                acc.at[pl.ds(bi * _CB, _CB)],
                out_hbm.at[pl.ds(c * _N + bi * _CB, _CB)],
            )


_sc_edge = functools.partial(
    pl.kernel,
    out_type=jax.ShapeDtypeStruct((_NC * _N, _D), jnp.float32),
    mesh=plsc.VectorSubcoreMesh(core_axis_name="c", subcore_axis_name="s"),
    scratch_types=(
        [
            pltpu.VMEM((_C,), jnp.int32),
            pltpu.VMEM((_C,), jnp.int32),
            pltpu.VMEM((_C,), jnp.int32),
            pltpu.VMEM((_C, _D), jnp.float32),
            pltpu.VMEM((_C, _D), jnp.int32),
            pltpu.VMEM((_C, _D), jnp.float32),
        ] * 3
        + [pltpu.VMEM_SHARED((_N, _D), jnp.float32)]
        + [pltpu.SemaphoreType.DMA] * 12
    ),
)(_sc_edge_body)


def kernel(x, edge_index, batch, W_enc, b_enc, Wk, bk, Wq, bq, Wv, bv,
           Wskip, bias):
    src = edge_index[0].astype(jnp.int32)
    dst = edge_index[1].astype(jnp.int32)
    batch32 = batch.astype(jnp.int32)

    wqvt = [jnp.concatenate([Wq[l].T, Wv[l].T], axis=1) for l in range(_L)]
    bqv = [jnp.concatenate([bq[l], bv[l]]) for l in range(_L)]

    k, qv, h = _enc_kqv(x, W_enc.T, b_enc, Wk[0].T, bk[0], wqvt[0], bqv[0])
    for l in range(_L - 1):
        parts = _sc_edge(k, qv, src, dst)
        k, qv, h = _out_kqv(
            h, Wskip[l].T, bias[l], parts.reshape(_NC, _N, _D),
            Wk[l + 1].T, bk[l + 1], wqvt[l + 1], bqv[l + 1],
        )
    parts = _sc_edge(k, qv, src, dst)
    return _out_pool(
        h, Wskip[_L - 1].T, bias[_L - 1], parts.reshape(_NC, _N, _D), batch32
    )
